# baseline (speedup 1.0000x reference)
import functools

import jax
import jax.numpy as jnp
from jax import lax
from jax.experimental import pallas as pl
from jax.experimental.pallas import tpu as pltpu
from jax.experimental.pallas import tpu_sc as plsc

FIELDS = 26
BATCH = 16384
DIM = 64
NUM_CORES = 2
NUM_SUBCORES = 16
NUM_WORKERS = NUM_CORES * NUM_SUBCORES
NLOOK = FIELDS * BATCH

_mesh = plsc.VectorSubcoreMesh(core_axis_name="c", subcore_axis_name="s")


G_CHUNK = 416
G_NBUF = 4
G_K = 2
G_PER_W = NLOOK // NUM_WORKERS
G_NCHUNK = G_PER_W // G_CHUNK


@functools.partial(
    pl.kernel,
    mesh=_mesh,
    out_type=jax.ShapeDtypeStruct((NLOOK, DIM), jnp.float32),
    scratch_types=[
        pltpu.VMEM((G_PER_W,), jnp.int32),
        [pltpu.VMEM((G_CHUNK, DIM), jnp.float32) for _ in range(G_NBUF)],
        [pltpu.SemaphoreType.DMA for _ in range(G_NBUF)],
        [pltpu.SemaphoreType.DMA for _ in range(G_NBUF)],
    ],
    compiler_params=pltpu.CompilerParams(use_tc_tiling_on_sc=False),
)
def _flat_gather(idx_hbm, table_hbm, out_hbm, idx_v, rows, gsem, wsem):
    wid = lax.axis_index("s") * jnp.int32(NUM_CORES) + lax.axis_index("c")
    wbase = pl.multiple_of(wid * jnp.int32(G_PER_W), G_CHUNK)
    pltpu.sync_copy(idx_hbm.at[pl.ds(wbase, G_PER_W)], idx_v)

    n_iters = G_NCHUNK + G_K
    n_outer = -(-n_iters // G_NBUF)

    def outer(c, carry):
        for b in range(G_NBUF):
            g = c * jnp.int32(G_NBUF) + jnp.int32(b)

            @pl.when(jnp.logical_and(g >= G_NBUF, g < G_NCHUNK))
            def _():
                pltpu.make_async_copy(
                    rows[b], out_hbm.at[pl.ds(wbase, G_CHUNK)], wsem[b]
                ).wait()

            @pl.when(g < G_NCHUNK)
            def _():
                off = pl.multiple_of(g * jnp.int32(G_CHUNK), G_CHUNK)
                pltpu.make_async_copy(
                    table_hbm.at[idx_v.at[pl.ds(off, G_CHUNK)]],
                    rows[b], gsem[b],
                ).start()

            h = g - jnp.int32(G_K)
            bh = (b - G_K) % G_NBUF

            @pl.when(jnp.logical_and(h >= 0, h < G_NCHUNK))
            def _():
                pltpu.make_async_copy(
                    table_hbm.at[idx_v.at[pl.ds(jnp.int32(0), G_CHUNK)]],
                    rows[bh], gsem[bh],
                ).wait()
                hoff = pl.multiple_of(
                    wbase + h * jnp.int32(G_CHUNK), G_CHUNK)
                pltpu.make_async_copy(
                    rows[bh], out_hbm.at[pl.ds(hoff, G_CHUNK)], wsem[bh]
                ).start()
        return carry

    lax.fori_loop(jnp.int32(0), jnp.int32(n_outer), outer, jnp.int32(0))

    for b in range(G_NBUF):
        pltpu.make_async_copy(
            rows[b], out_hbm.at[pl.ds(wbase, G_CHUNK)], wsem[b]
        ).wait()


T_CHUNK = 256
T_UNITS = NLOOK // T_CHUNK
T_BPW = T_UNITS // NUM_WORKERS
T_PROWS = T_CHUNK // 2


@functools.partial(
    pl.kernel,
    mesh=_mesh,
    out_type=jax.ShapeDtypeStruct((FIELDS * DIM, BATCH), jnp.float32),
    scratch_types=[
        [pltpu.VMEM((T_PROWS, 128), jnp.float32) for _ in range(2)],
        [pltpu.VMEM((DIM, T_CHUNK), jnp.float32) for _ in range(2)],
        [pltpu.SemaphoreType.DMA for _ in range(2)],
        [pltpu.SemaphoreType.DMA for _ in range(2)],
    ],
    compiler_params=pltpu.CompilerParams(
        use_tc_tiling_on_sc=True, needs_layout_passes=False),
)
def _to_native(inter_hbm, out_hbm, sbuf, obuf, rsem, wsem):
    wid = lax.axis_index("s") * jnp.int32(NUM_CORES) + lax.axis_index("c")

    iota16 = lax.iota(jnp.int32, 16)
    rot = [lax.bitwise_and(iota16 + jnp.int32(k), jnp.int32(15))
           for k in range(16)]
    half_iota = lax.shift_right_logical(iota16, jnp.int32(1))
    odd64 = lax.bitwise_and(iota16, jnp.int32(1)) * jnp.int32(DIM)

    def in_slice(t):
        u0 = wid * jnp.int32(T_BPW) + t
        r0 = pl.multiple_of(u0 * jnp.int32(T_PROWS), T_PROWS)
        return inter_hbm.at[pl.ds(r0, T_PROWS), :]

    def out_slice(t):
        u0 = wid * jnp.int32(T_BPW) + t
        f = u0 // jnp.int32(BATCH // T_CHUNK)
        j = u0 % jnp.int32(BATCH // T_CHUNK)
        row0 = pl.multiple_of(f * jnp.int32(DIM), DIM)
        col0 = pl.multiple_of(j * jnp.int32(T_CHUNK), T_CHUNK)
        return out_hbm.at[pl.ds(row0, DIM), pl.ds(col0, T_CHUNK)]

    def read_start(t, b):
        pltpu.make_async_copy(in_slice(t), sbuf[b], rsem[b]).start()

    def read_wait(t, b):
        pltpu.make_async_copy(in_slice(t), sbuf[b], rsem[b]).wait()

    def write_start(t, b):
        pltpu.make_async_copy(obuf[b], out_slice(t), wsem[b]).start()

    def write_wait(t, b):
        pltpu.make_async_copy(obuf[b], out_slice(t), wsem[b]).wait()

    def transpose(b):
        def group(g, carry):
            g16 = g * jnp.int32(16)
            bvec = iota16 + g16
            rvec = half_iota + g * jnp.int32(8)
            for d0 in range(0, DIM, 16):
                for k in range(16):
                    rd = rot[k] + jnp.int32(d0)
                    w = plsc.load_gather(sbuf[b], [rvec, odd64 + rd])
                    plsc.store_scatter(obuf[b], [rd, bvec], w)
            return carry

        lax.fori_loop(jnp.int32(0), jnp.int32(T_CHUNK // 16), group,
                      jnp.int32(0))

    read_start(jnp.int32(0), 0)

    def outer(c, carry):
        for b in range(2):
            t = c * jnp.int32(2) + jnp.int32(b)

            @pl.when(t + jnp.int32(1) < jnp.int32(T_BPW))
            def _():
                read_start(t + jnp.int32(1), 1 - b)

            read_wait(t, b)

            @pl.when(t >= jnp.int32(2))
            def _():
                write_wait(t - jnp.int32(2), b)

            transpose(b)
            write_start(t, b)
        return carry

    lax.fori_loop(jnp.int32(0), jnp.int32(T_BPW // 2), outer, jnp.int32(0))

    write_wait(jnp.int32(T_BPW - 2), 0)
    write_wait(jnp.int32(T_BPW - 1), 1)


def kernel(keys, table):
    flat = keys.T.reshape(-1).astype(jnp.int32)
    rows = _flat_gather(flat, table)
    inter = rows.reshape(NLOOK // 2, 128)
    out_p = _to_native(inter)
    return out_p.reshape(FIELDS, DIM, BATCH).transpose(2, 0, 1)


# speedup vs baseline: 1.0216x; 1.0216x over previous
import functools

import jax
import jax.numpy as jnp
from jax import lax
from jax.experimental import pallas as pl
from jax.experimental.pallas import tpu as pltpu
from jax.experimental.pallas import tpu_sc as plsc

FIELDS = 26
BATCH = 16384
DIM = 64
NUM_CORES = 2
NUM_SUBCORES = 16
NUM_WORKERS = NUM_CORES * NUM_SUBCORES
NLOOK = FIELDS * BATCH

_mesh = plsc.VectorSubcoreMesh(core_axis_name="c", subcore_axis_name="s")


G_CHUNK = 208
G_NBUF = 3
G_K = 2
G_PER_W = NLOOK // NUM_WORKERS
G_NCHUNK = G_PER_W // G_CHUNK


@functools.partial(
    pl.kernel,
    mesh=_mesh,
    out_type=jax.ShapeDtypeStruct((NLOOK, DIM), jnp.float32),
    scratch_types=[
        pltpu.VMEM((G_PER_W,), jnp.int32),
        [pltpu.VMEM((G_CHUNK, 128), jnp.float32) for _ in range(G_NBUF)],
        [pltpu.SemaphoreType.DMA for _ in range(G_NBUF)],
        [pltpu.SemaphoreType.DMA for _ in range(G_NBUF)],
    ],
    compiler_params=pltpu.CompilerParams(use_tc_tiling_on_sc=False),
)
def _flat_gather(idx_hbm, table_hbm, out_hbm, idx_v, rows, gsem, wsem):
    wid = lax.axis_index("s") * jnp.int32(NUM_CORES) + lax.axis_index("c")
    wbase = pl.multiple_of(wid * jnp.int32(G_PER_W), G_CHUNK)
    pltpu.sync_copy(idx_hbm.at[pl.ds(wbase, G_PER_W)], idx_v)

    n_iters = G_NCHUNK + G_K
    n_outer = -(-n_iters // G_NBUF)

    def outer(c, carry):
        for b in range(G_NBUF):
            g = c * jnp.int32(G_NBUF) + jnp.int32(b)

            @pl.when(jnp.logical_and(g >= G_NBUF, g < G_NCHUNK))
            def _():
                pltpu.make_async_copy(
                    rows[b].at[:, pl.ds(jnp.int32(0), DIM)],
                    out_hbm.at[pl.ds(wbase, G_CHUNK)], wsem[b]
                ).wait()

            @pl.when(g < G_NCHUNK)
            def _():
                off = pl.multiple_of(g * jnp.int32(G_CHUNK), G_CHUNK)
                pltpu.make_async_copy(
                    table_hbm.at[idx_v.at[pl.ds(off, G_CHUNK)]],
                    rows[b], gsem[b],
                ).start()

            h = g - jnp.int32(G_K)
            bh = (b - G_K) % G_NBUF

            @pl.when(jnp.logical_and(h >= 0, h < G_NCHUNK))
            def _():
                pltpu.make_async_copy(
                    table_hbm.at[idx_v.at[pl.ds(jnp.int32(0), G_CHUNK)]],
                    rows[bh], gsem[bh],
                ).wait()
                hoff = pl.multiple_of(
                    wbase + h * jnp.int32(G_CHUNK), G_CHUNK)
                pltpu.make_async_copy(
                    rows[bh].at[:, pl.ds(jnp.int32(0), DIM)],
                    out_hbm.at[pl.ds(hoff, G_CHUNK)], wsem[bh]
                ).start()
        return carry

    lax.fori_loop(jnp.int32(0), jnp.int32(n_outer), outer, jnp.int32(0))

    for b in range(G_NBUF):
        pltpu.make_async_copy(
            rows[b].at[:, pl.ds(jnp.int32(0), DIM)],
            out_hbm.at[pl.ds(wbase, G_CHUNK)], wsem[b]
        ).wait()


T_CHUNK = 256
T_UNITS = NLOOK // T_CHUNK
T_BPW = T_UNITS // NUM_WORKERS
T_PROWS = T_CHUNK // 2


@functools.partial(
    pl.kernel,
    mesh=_mesh,
    out_type=jax.ShapeDtypeStruct((FIELDS * DIM, BATCH), jnp.float32),
    scratch_types=[
        [pltpu.VMEM((T_PROWS, 128), jnp.float32) for _ in range(2)],
        [pltpu.VMEM((DIM, T_CHUNK), jnp.float32) for _ in range(2)],
        [pltpu.SemaphoreType.DMA for _ in range(2)],
        [pltpu.SemaphoreType.DMA for _ in range(2)],
    ],
    compiler_params=pltpu.CompilerParams(
        use_tc_tiling_on_sc=True, needs_layout_passes=False),
)
def _to_native(inter_hbm, out_hbm, sbuf, obuf, rsem, wsem):
    wid = lax.axis_index("s") * jnp.int32(NUM_CORES) + lax.axis_index("c")

    iota16 = lax.iota(jnp.int32, 16)
    rot = [lax.bitwise_and(iota16 + jnp.int32(k), jnp.int32(15))
           for k in range(16)]
    half_iota = lax.shift_right_logical(iota16, jnp.int32(1))
    odd64 = lax.bitwise_and(iota16, jnp.int32(1)) * jnp.int32(DIM)

    def in_slice(t):
        u0 = wid * jnp.int32(T_BPW) + t
        r0 = pl.multiple_of(u0 * jnp.int32(T_PROWS), T_PROWS)
        return inter_hbm.at[pl.ds(r0, T_PROWS), :]

    def out_slice(t):
        u0 = wid * jnp.int32(T_BPW) + t
        f = u0 // jnp.int32(BATCH // T_CHUNK)
        j = u0 % jnp.int32(BATCH // T_CHUNK)
        row0 = pl.multiple_of(f * jnp.int32(DIM), DIM)
        col0 = pl.multiple_of(j * jnp.int32(T_CHUNK), T_CHUNK)
        return out_hbm.at[pl.ds(row0, DIM), pl.ds(col0, T_CHUNK)]

    def read_start(t, b):
        pltpu.make_async_copy(in_slice(t), sbuf[b], rsem[b]).start()

    def read_wait(t, b):
        pltpu.make_async_copy(in_slice(t), sbuf[b], rsem[b]).wait()

    def write_start(t, b):
        pltpu.make_async_copy(obuf[b], out_slice(t), wsem[b]).start()

    def write_wait(t, b):
        pltpu.make_async_copy(obuf[b], out_slice(t), wsem[b]).wait()

    def transpose(b):
        def group(g, carry):
            g16 = g * jnp.int32(16)
            bvec = iota16 + g16
            rvec = half_iota + g * jnp.int32(8)
            for d0 in range(0, DIM, 16):
                for k in range(16):
                    rd = rot[k] + jnp.int32(d0)
                    w = plsc.load_gather(sbuf[b], [rvec, odd64 + rd])
                    plsc.store_scatter(obuf[b], [rd, bvec], w)
            return carry

        lax.fori_loop(jnp.int32(0), jnp.int32(T_CHUNK // 16), group,
                      jnp.int32(0))

    read_start(jnp.int32(0), 0)

    def outer(c, carry):
        for b in range(2):
            t = c * jnp.int32(2) + jnp.int32(b)

            @pl.when(t + jnp.int32(1) < jnp.int32(T_BPW))
            def _():
                read_start(t + jnp.int32(1), 1 - b)

            read_wait(t, b)

            @pl.when(t >= jnp.int32(2))
            def _():
                write_wait(t - jnp.int32(2), b)

            transpose(b)
            write_start(t, b)
        return carry

    lax.fori_loop(jnp.int32(0), jnp.int32(T_BPW // 2), outer, jnp.int32(0))

    write_wait(jnp.int32(T_BPW - 2), 0)
    write_wait(jnp.int32(T_BPW - 1), 1)


def kernel(keys, table):
    flat = keys.T.reshape(-1).astype(jnp.int32)
    tbl_p = jnp.pad(table, ((0, 0), (0, 128 - DIM)))
    rows = _flat_gather(flat, tbl_p)
    inter = rows.reshape(NLOOK // 2, 128)
    out_p = _to_native(inter)
    return out_p.reshape(FIELDS, DIM, BATCH).transpose(2, 0, 1)


# speedup vs baseline: 1.1099x; 1.0865x over previous
import functools

import jax
import jax.numpy as jnp
from jax import lax
from jax.experimental import pallas as pl
from jax.experimental.pallas import tpu as pltpu
from jax.experimental.pallas import tpu_sc as plsc

FIELDS = 26
BATCH = 16384
DIM = 64
NUM_CORES = 2
NUM_SUBCORES = 16
NUM_WORKERS = NUM_CORES * NUM_SUBCORES

CHUNK = 128
UNITS = FIELDS * (BATCH // CHUNK)
BPW = UNITS // NUM_WORKERS
IDX_PER_W = BPW * CHUNK
NBUF = 4

_mesh = plsc.VectorSubcoreMesh(core_axis_name="c", subcore_axis_name="s")


@functools.partial(
    pl.kernel,
    mesh=_mesh,
    out_type=jax.ShapeDtypeStruct((FIELDS * DIM, BATCH), jnp.float32),
    scratch_types=[
        pltpu.VMEM((IDX_PER_W,), jnp.int32),
        [pltpu.VMEM((CHUNK,), jnp.int32) for _ in range(NBUF)],
        [pltpu.VMEM((CHUNK,), jnp.int32) for _ in range(NBUF)],
        [pltpu.VMEM((CHUNK, 128), jnp.float32) for _ in range(NBUF)],
        [pltpu.VMEM((DIM, CHUNK), jnp.float32) for _ in range(2)],
        [pltpu.SemaphoreType.DMA for _ in range(NBUF)],
        [pltpu.SemaphoreType.DMA for _ in range(2)],
    ],
    compiler_params=pltpu.CompilerParams(
        use_tc_tiling_on_sc=True, needs_layout_passes=False),
)
def _sc_gather(idx_hbm, tbl_hbm, out_hbm, idxbuf, qbuf, parbuf, gbuf, obuf,
               gsem, wsem):
    wid = lax.axis_index("s") * jnp.int32(NUM_CORES) + lax.axis_index("c")
    wbase = pl.multiple_of(wid * jnp.int32(IDX_PER_W), CHUNK)
    pltpu.sync_copy(idx_hbm.at[pl.ds(wbase, IDX_PER_W)], idxbuf)

    iota16 = lax.iota(jnp.int32, 16)
    rot = [lax.bitwise_and(iota16 + jnp.int32(k), jnp.int32(15))
           for k in range(16)]

    def prep(t, g):
        for v in range(CHUNK // 16):
            x = idxbuf[pl.ds(t * jnp.int32(CHUNK) + jnp.int32(v * 16), 16)]
            qbuf[g][pl.ds(jnp.int32(v * 16), 16)] = lax.shift_right_logical(
                x, jnp.int32(1))
            parbuf[g][pl.ds(jnp.int32(v * 16), 16)] = lax.bitwise_and(
                x, jnp.int32(1))

    def gather_start(g):
        pltpu.make_async_copy(tbl_hbm.at[qbuf[g]], gbuf[g], gsem[g]).start()

    def gather_wait(g):
        pltpu.make_async_copy(tbl_hbm.at[qbuf[g]], gbuf[g], gsem[g]).wait()

    def batch_out_slice(t):
        u0 = wid * jnp.int32(BPW) + t
        f = u0 // jnp.int32(BATCH // CHUNK)
        j = u0 % jnp.int32(BATCH // CHUNK)
        row0 = pl.multiple_of(f * jnp.int32(DIM), DIM)
        col0 = pl.multiple_of(j * jnp.int32(CHUNK), CHUNK)
        return out_hbm.at[pl.ds(row0, DIM), pl.ds(col0, CHUNK)]

    def transpose(g, ob):
        def group(gr, carry):
            g16 = gr * jnp.int32(16)
            bvec = iota16 + g16
            parv = parbuf[g][pl.ds(g16, 16)] * jnp.int32(DIM)
            for d0 in range(0, DIM, 16):
                for k in range(16):
                    rd = rot[k] + jnp.int32(d0)
                    w = plsc.load_gather(gbuf[g], [bvec, parv + rd])
                    plsc.store_scatter(obuf[ob], [rd, bvec], w)
            return carry

        lax.fori_loop(jnp.int32(0), jnp.int32(CHUNK // 16), group,
                      jnp.int32(0))

    def write_start(t, ob):
        pltpu.make_async_copy(obuf[ob], batch_out_slice(t), wsem[ob]).start()

    def write_wait(t, ob):
        pltpu.make_async_copy(obuf[ob], batch_out_slice(t), wsem[ob]).wait()

    for g in range(NBUF - 1):
        prep(jnp.int32(g), g)
        gather_start(g)

    def outer(c, carry):
        for b in range(NBUF):
            t = c * jnp.int32(NBUF) + jnp.int32(b)
            ob = b % 2

            @pl.when(t + jnp.int32(NBUF - 1) < jnp.int32(BPW))
            def _():
                prep(t + jnp.int32(NBUF - 1), (b + NBUF - 1) % NBUF)
                gather_start((b + NBUF - 1) % NBUF)

            gather_wait(b)

            @pl.when(t >= jnp.int32(2))
            def _():
                write_wait(t - jnp.int32(2), ob)

            transpose(b, ob)
            write_start(t, ob)
        return carry

    lax.fori_loop(jnp.int32(0), jnp.int32(BPW // NBUF), outer, jnp.int32(0))

    write_wait(jnp.int32(BPW - 2), 0)
    write_wait(jnp.int32(BPW - 1), 1)


def kernel(keys, table):
    flat = keys.T.reshape(-1).astype(jnp.int32)
    tbl = table.reshape(500000, 128)
    out_p = _sc_gather(flat, tbl)
    return out_p.reshape(FIELDS, DIM, BATCH).transpose(2, 0, 1)
